# baseline (speedup 1.0000x reference)
import functools

import jax
import jax.numpy as jnp
from jax import lax
from jax.experimental import pallas as pl
from jax.experimental.pallas import tpu as pltpu
from jax.experimental.pallas import tpu_sc as plsc

N_ROWS = 100000
D = 128
NUM_CORES = 2
NUM_SUBCORES = 16
NW = NUM_CORES * NUM_SUBCORES
SPAN = 3128
CHUNK = 400
NFULL = 7
TAIL_A = SPAN - NFULL * CHUNK
TAIL_B = (N_ROWS - (NW - 1) * SPAN) - NFULL * CHUNK
STAGE = 3328
STAGE_LAST = 3200

_mesh = plsc.VectorSubcoreMesh(core_axis_name="c", subcore_axis_name="s")


@functools.partial(
    pl.kernel,
    mesh=_mesh,
    out_type=jax.ShapeDtypeStruct((N_ROWS, D), jnp.float32),
    scratch_types=[
        pltpu.VMEM((STAGE,), jnp.int32),
        pltpu.VMEM((2, CHUNK, D), jnp.float32),
        pltpu.SemaphoreType.DMA((2,)),
        pltpu.SemaphoreType.DMA((2,)),
    ],
)
def _gather(table_hbm, idx_hbm, out_hbm, idx_v, rows_v, gsem, wsem):
    wid = lax.axis_index("s") * NUM_CORES + lax.axis_index("c")
    base = wid * SPAN
    a0 = pl.multiple_of(base - lax.rem(base, 128), 128)
    delta = base - a0

    @pl.when(wid < NW - 1)
    def _():
        pltpu.sync_copy(idx_hbm.at[0, pl.ds(a0, STAGE)], idx_v)

    @pl.when(wid == NW - 1)
    def _():
        pltpu.sync_copy(idx_hbm.at[0, pl.ds(a0, STAGE_LAST)],
                        idx_v.at[pl.ds(0, STAGE_LAST)])

    def start_gather(j):
        b = lax.rem(j, 2)
        pltpu.async_copy(
            table_hbm.at[idx_v.at[pl.ds(delta + j * CHUNK, CHUNK)]],
            rows_v.at[b], gsem.at[b])

    def tail_descr(tail):
        return pltpu.make_async_copy(
            table_hbm.at[idx_v.at[pl.ds(delta + NFULL * CHUNK, tail)]],
            rows_v.at[1, pl.ds(0, tail)], gsem.at[1])

    def start_tail_gather():
        @pl.when(wid < NW - 1)
        def _():
            tail_descr(TAIL_A).start()

        @pl.when(wid == NW - 1)
        def _():
            tail_descr(TAIL_B).start()

    def wait_write(j):
        b = lax.rem(j, 2)
        pltpu.make_async_copy(
            rows_v.at[b], out_hbm.at[pl.ds(base + j * CHUNK, CHUNK)],
            wsem.at[b]).wait()

    start_gather(0)

    def step(j, carry):
        b = lax.rem(j, 2)

        @pl.when(j >= 1)
        def _():
            wait_write(j - 1)

        @pl.when(j + 1 < NFULL)
        def _():
            start_gather(j + 1)

        @pl.when(j + 1 == NFULL)
        def _():
            start_tail_gather()

        pltpu.make_async_copy(
            table_hbm.at[idx_v.at[pl.ds(delta + j * CHUNK, CHUNK)]],
            rows_v.at[b], gsem.at[b]).wait()
        pltpu.async_copy(
            rows_v.at[b], out_hbm.at[pl.ds(base + j * CHUNK, CHUNK)],
            wsem.at[b])
        return carry

    lax.fori_loop(0, NFULL, step, 0)

    wait_write(NFULL - 1)
    toff = base + NFULL * CHUNK

    @pl.when(wid < NW - 1)
    def _():
        tail_descr(TAIL_A).wait()
        pltpu.async_copy(rows_v.at[1, pl.ds(0, TAIL_A)],
                         out_hbm.at[pl.ds(toff, TAIL_A)], wsem.at[1])
        pltpu.make_async_copy(rows_v.at[1, pl.ds(0, TAIL_A)],
                              out_hbm.at[pl.ds(toff, TAIL_A)],
                              wsem.at[1]).wait()

    @pl.when(wid == NW - 1)
    def _():
        tail_descr(TAIL_B).wait()
        pltpu.async_copy(rows_v.at[1, pl.ds(0, TAIL_B)],
                         out_hbm.at[pl.ds(toff, TAIL_B)], wsem.at[1])
        pltpu.make_async_copy(rows_v.at[1, pl.ds(0, TAIL_B)],
                              out_hbm.at[pl.ds(toff, TAIL_B)],
                              wsem.at[1]).wait()


def kernel(g, h, r, norm, W):
    idx = h.reshape(1, -1).astype(jnp.int32)
    return _gather(W, idx)


# speedup vs baseline: 1.0049x; 1.0049x over previous
import functools

import jax
import jax.numpy as jnp
from jax import lax
from jax.experimental import pallas as pl
from jax.experimental.pallas import tpu as pltpu
from jax.experimental.pallas import tpu_sc as plsc

N_ROWS = 100000
D = 128
NUM_CORES = 2
NUM_SUBCORES = 16
NW = NUM_CORES * NUM_SUBCORES
SPAN = 3200
CHUNK = 400
NFULL = SPAN // CHUNK
NLAST = (N_ROWS - (NW - 1) * SPAN) // CHUNK

_mesh = plsc.VectorSubcoreMesh(core_axis_name="c", subcore_axis_name="s")


@functools.partial(
    pl.kernel,
    mesh=_mesh,
    out_type=jax.ShapeDtypeStruct((N_ROWS, D), jnp.float32),
    scratch_types=[
        pltpu.VMEM((SPAN,), jnp.int32),
        pltpu.VMEM((2, CHUNK, D), jnp.float32),
        pltpu.SemaphoreType.DMA((2,)),
        pltpu.SemaphoreType.DMA((2,)),
    ],
)
def _gather(table_hbm, idx_hbm, out_hbm, idx_v, rows_v, gsem, wsem):
    wid = lax.axis_index("s") * NUM_CORES + lax.axis_index("c")
    base = wid * SPAN
    nch = jnp.where(wid == NW - 1, NLAST, NFULL)

    @pl.when(wid < NW - 1)
    def _():
        pltpu.sync_copy(idx_hbm.at[0, pl.ds(base, SPAN)], idx_v)

    @pl.when(wid == NW - 1)
    def _():
        pltpu.sync_copy(idx_hbm.at[0, pl.ds(base, 896)],
                        idx_v.at[pl.ds(0, 896)])

    def start_gather(j):
        b = lax.rem(j, 2)
        pltpu.async_copy(
            table_hbm.at[idx_v.at[pl.ds(j * CHUNK, CHUNK)]],
            rows_v.at[b], gsem.at[b])

    def wait_write(j):
        b = lax.rem(j, 2)
        pltpu.make_async_copy(
            rows_v.at[b], out_hbm.at[pl.ds(base + j * CHUNK, CHUNK)],
            wsem.at[b]).wait()

    start_gather(0)

    def step(j, carry):
        b = lax.rem(j, 2)

        @pl.when(j + 1 < nch)
        def _():
            @pl.when(j >= 1)
            def _():
                wait_write(j - 1)
            start_gather(j + 1)

        pltpu.make_async_copy(
            table_hbm.at[idx_v.at[pl.ds(j * CHUNK, CHUNK)]],
            rows_v.at[b], gsem.at[b]).wait()
        pltpu.async_copy(
            rows_v.at[b], out_hbm.at[pl.ds(base + j * CHUNK, CHUNK)],
            wsem.at[b])
        return carry

    lax.fori_loop(0, nch, step, 0)
    wait_write(nch - 2)
    wait_write(nch - 1)


def kernel(g, h, r, norm, W):
    idx = h.reshape(1, -1).astype(jnp.int32)
    return _gather(W, idx)
